# Initial kernel scaffold; baseline (speedup 1.0000x reference)
import jax
import jax.numpy as jnp
from jax.experimental import pallas as pl


def kernel(output, xt):
    raise NotImplementedError("write your pallas kernel here")


# speedup vs baseline: 1.9744x; 1.9744x over previous
import jax
import jax.numpy as jnp
from jax.experimental import pallas as pl

_B, _L, _V = 32, 32, 32001
_MASK_IDX = 32000
_N = _B * _L
_R = 16


def _body(xt_ref, x_ref, o_ref):
    x = x_ref[...]
    xt = xt_ref[...]
    lane = jax.lax.broadcasted_iota(jnp.int32, (_R, _V), 1)
    neg_inf = jnp.float32(-jnp.inf)
    valid = lane < _MASK_IDX
    xm = jnp.where(valid, x, neg_inf)
    m = jnp.max(xm, axis=-1, keepdims=True)
    s = jnp.sum(jnp.exp(xm - m), axis=-1, keepdims=True)
    lse = m + jnp.log(s)
    sm = jnp.where(valid, x - lse, neg_inf)
    onehot = jnp.where(lane == xt, jnp.float32(0.0), neg_inf)
    unmasked = xt != _MASK_IDX
    o_ref[...] = jnp.where(unmasked, onehot, sm)


def kernel(output, xt):
    x = output.reshape(_N, _V)
    xt2 = xt.reshape(_N, 1)
    out = pl.pallas_call(
        _body,
        grid=(_N // _R,),
        in_specs=[
            pl.BlockSpec((_R, 1), lambda i: (i, 0)),
            pl.BlockSpec((_R, _V), lambda i: (i, 0)),
        ],
        out_specs=pl.BlockSpec((_R, _V), lambda i: (i, 0)),
        out_shape=jax.ShapeDtypeStruct((_N, _V), jnp.float32),
    )(xt2, x)
    return out.reshape(_B, _L, _V)


# speedup vs baseline: 3.6964x; 1.8721x over previous
import jax
import jax.numpy as jnp
from jax.experimental import pallas as pl
from jax.experimental.pallas import tpu as pltpu

_B, _L, _V = 32, 32, 32001
_MASK_IDX = 32000
_N = _B * _L
_R = 16
_NB = _N // _R


def _body(src_ref, flag_ref, xt_ref, x_ref, o_ref):
    i = pl.program_id(0)
    xt = xt_ref[...]
    lane = jax.lax.broadcasted_iota(jnp.int32, (_R, _V), 1)
    neg_inf = jnp.float32(-jnp.inf)
    onehot = jnp.where(lane == xt, jnp.float32(0.0), neg_inf)

    @pl.when(flag_ref[i] == 0)
    def _no_masked_rows():
        o_ref[...] = onehot

    @pl.when(flag_ref[i] != 0)
    def _has_masked_rows():
        x = x_ref[...]
        valid = lane < _MASK_IDX
        xm = jnp.where(valid, x, neg_inf)
        m = jnp.max(xm, axis=-1, keepdims=True)
        s = jnp.sum(jnp.exp(xm - m), axis=-1, keepdims=True)
        lse = m + jnp.log(s)
        sm = jnp.where(valid, x - lse, neg_inf)
        o_ref[...] = jnp.where(xt != _MASK_IDX, onehot, sm)


def kernel(output, xt):
    x = output.reshape(_N, _V)
    xt2 = xt.reshape(_N, 1)
    blk_has = jnp.any(
        (xt2[:, 0] == _MASK_IDX).reshape(_NB, _R), axis=1)
    flags = blk_has.astype(jnp.int32)
    src = jax.lax.cummax(
        jnp.where(blk_has, jnp.arange(_NB, dtype=jnp.int32), 0))
    grid_spec = pltpu.PrefetchScalarGridSpec(
        num_scalar_prefetch=2,
        grid=(_NB,),
        in_specs=[
            pl.BlockSpec((_R, 1), lambda i, src_ref, flag_ref: (i, 0)),
            pl.BlockSpec((_R, _V),
                         lambda i, src_ref, flag_ref: (src_ref[i], 0)),
        ],
        out_specs=pl.BlockSpec((_R, _V), lambda i, src_ref, flag_ref: (i, 0)),
    )
    out = pl.pallas_call(
        _body,
        grid_spec=grid_spec,
        out_shape=jax.ShapeDtypeStruct((_N, _V), jnp.float32),
    )(src, flags, xt2, x)
    return out.reshape(_B, _L, _V)


# speedup vs baseline: 4.8788x; 1.3199x over previous
import jax
import jax.numpy as jnp
from jax.experimental import pallas as pl
from jax.experimental.pallas import tpu as pltpu

_B, _L, _V = 32, 32, 32001
_MASK_IDX = 32000
_N = _B * _L
_R = 32
_NB = _N // _R


def _body(src_ref, flag_ref, xt_ref, x_ref, o_ref):
    i = pl.program_id(0)
    xt = xt_ref[...]
    lane = jax.lax.broadcasted_iota(jnp.int32, (_R, _V), 1)
    neg_inf = jnp.float32(-jnp.inf)
    onehot = jnp.where(lane == xt, jnp.float32(0.0), neg_inf)

    @pl.when(flag_ref[i] == 0)
    def _no_masked_rows():
        o_ref[...] = onehot

    @pl.when(flag_ref[i] != 0)
    def _has_masked_rows():
        x = x_ref[...]
        valid = lane < _MASK_IDX
        xm = jnp.where(valid, x, neg_inf)
        m = jnp.max(xm, axis=-1, keepdims=True)
        s = jnp.sum(jnp.exp(xm - m), axis=-1, keepdims=True)
        lse = m + jnp.log(s)
        sm = jnp.where(valid, x - lse, neg_inf)
        o_ref[...] = jnp.where(xt != _MASK_IDX, onehot, sm)


def kernel(output, xt):
    x = output.reshape(_N, _V)
    xt2 = xt.reshape(_N, 1)
    blk_has = jnp.any(
        (xt2[:, 0] == _MASK_IDX).reshape(_NB, _R), axis=1)
    flags = blk_has.astype(jnp.int32)
    src = jax.lax.cummax(
        jnp.where(blk_has, jnp.arange(_NB, dtype=jnp.int32), 0))
    grid_spec = pltpu.PrefetchScalarGridSpec(
        num_scalar_prefetch=2,
        grid=(_NB,),
        in_specs=[
            pl.BlockSpec((_R, 1), lambda i, src_ref, flag_ref: (i, 0)),
            pl.BlockSpec((_R, _V),
                         lambda i, src_ref, flag_ref: (src_ref[i], 0)),
        ],
        out_specs=pl.BlockSpec((_R, _V), lambda i, src_ref, flag_ref: (i, 0)),
    )
    out = pl.pallas_call(
        _body,
        grid_spec=grid_spec,
        out_shape=jax.ShapeDtypeStruct((_N, _V), jnp.float32),
    )(src, flags, xt2, x)
    return out.reshape(_B, _L, _V)
